# baseline (speedup 1.0000x reference)
import jax
import jax.numpy as jnp
from jax import lax
from jax.experimental import pallas as pl
from jax.experimental.pallas import tpu as pltpu
from jax.experimental.pallas import tpu_sc as plsc

B = 4096
L = 200
D = 64
NC, NS = 2, 16
NW = NC * NS
LT = L // 8
BT = B // 128
BLOCKS = L * BT
BPW = BLOCKS // NW


def _body(ids_h, tab_h, pos_h, out_h, idsv, rowsv, obufv, posv, gsem, isem, psem, osem):
    cid = lax.axis_index("c")
    sid = lax.axis_index("s")
    wid = sid * NC + cid
    t0 = wid * BPW

    iota = lax.iota(jnp.int32, 16)

    def ids_src(t):
        l = t // BT
        bt = t % BT
        return ids_h.at[l // 8, bt, l % 8]

    def fire_ids(t, k):
        pltpu.async_copy(ids_src(t), idsv[k], isem[k])
        pltpu.async_copy(pos_h.at[t // BT], posv[k], psem[k])

    def wait_ids(t, k):
        pltpu.make_async_copy(ids_src(t), idsv[k], isem[k]).wait()
        pltpu.make_async_copy(pos_h.at[t // BT], posv[k], psem[k]).wait()

    def fire_gather(k):
        pltpu.async_copy(tab_h.at[idsv[k]], rowsv[k], gsem[k])

    def wait_gather(k):
        pltpu.make_async_copy(tab_h.at[idsv[k]], rowsv[k], gsem[k]).wait()

    def out_dst(t):
        return out_h.at[t // BT, :, t % BT]

    def fire_out(t, k):
        pltpu.async_copy(obufv[k], out_dst(t), osem[k])

    def wait_out(t, k):
        pltpu.make_async_copy(obufv[k], out_dst(t), osem[k]).wait()

    def compute(k):
        def dbody(d, c):
            pv = posv[k][d, pl.ds(0, 16)]
            dvec = jnp.full((16,), 0, jnp.int32) + d
            for j in range(8):
                rows = iota + (16 * j)
                val = plsc.load_gather(rowsv[k], [rows, dvec]) + pv
                obufv[k][d // 8, d % 8, pl.ds(16 * j, 16)] = val
            return c

        lax.fori_loop(0, D, dbody, 0)

    fire_ids(t0, 0)
    wait_ids(t0, 0)
    fire_gather(0)
    fire_ids(t0 + 1, 1)

    def stage(tt, k, fire_next_gather, fire_next_ids, wait_prev_out):
        t = t0 + tt
        wait_gather(k)
        if fire_next_gather:
            wait_ids(t + 1, 1 - k)
            fire_gather(1 - k)
        if wait_prev_out:
            wait_out(t - 2, k)
        compute(k)
        fire_out(t, k)
        if fire_next_ids:
            fire_ids(t + 2, k)

    stage(0, 0, True, True, False)
    stage(1, 1, True, True, False)

    def steady(i, c):
        tt = 2 * i + 2
        stage(tt, 0, True, True, True)
        stage(tt + 1, 1, True, True, True)
        return c

    lax.fori_loop(0, (BPW - 4) // 2, steady, 0)

    stage(BPW - 2, 0, True, False, True)
    stage(BPW - 1, 1, False, False, True)
    wait_out(t0 + BPW - 2, 0)
    wait_out(t0 + BPW - 1, 1)


@jax.jit
def _run(ids4d, table, posb):
    mesh = plsc.VectorSubcoreMesh(
        core_axis_name="c", subcore_axis_name="s", num_cores=NC, num_subcores=NS
    )
    f = pl.kernel(
        _body,
        out_type=jax.ShapeDtypeStruct((L, 8, BT, 8, 128), jnp.float32),
        mesh=mesh,
        scratch_types=[
            [pltpu.VMEM((128,), jnp.int32) for _ in range(2)],
            [pltpu.VMEM((128, D), jnp.float32) for _ in range(2)],
            [pltpu.VMEM((8, 8, 128), jnp.float32) for _ in range(2)],
            [pltpu.VMEM((D, 16), jnp.float32) for _ in range(2)],
            [pltpu.SemaphoreType.DMA for _ in range(2)],
            [pltpu.SemaphoreType.DMA for _ in range(2)],
            [pltpu.SemaphoreType.DMA for _ in range(2)],
            [pltpu.SemaphoreType.DMA for _ in range(2)],
        ],
        compiler_params=pltpu.CompilerParams(use_tc_tiling_on_sc=False, needs_layout_passes=False),
    )
    return f(ids4d, table, posb)


def kernel(input_ids, token_table, pos_table):
    ids4d = (
        input_ids.astype(jnp.int32)
        .T.reshape(LT, 8, BT, 128)
        .transpose(0, 2, 1, 3)
    )
    posb = jnp.broadcast_to(
        pos_table[:L].T.reshape(1, D, L).transpose(2, 1, 0), (L, D, 16)
    )
    out5d = _run(ids4d, token_table, posb)
    return out5d.transpose(2, 4, 0, 1, 3).reshape(B, L, D)


# speedup vs baseline: 1.0011x; 1.0011x over previous
import jax
import jax.numpy as jnp
from jax import lax
from jax.experimental import pallas as pl
from jax.experimental.pallas import tpu as pltpu
from jax.experimental.pallas import tpu_sc as plsc

B = 4096
L = 200
D = 64
NC, NS = 2, 16
NW = NC * NS
LT = L // 8
BT = B // 128
BLOCKS = L * BT
BPW = BLOCKS // NW
NB = 4


def _body(ids_h, tab_h, pos_h, out_h, idsv, rowsv, obufv, posv, gsem, isem, psem, osem):
    cid = lax.axis_index("c")
    sid = lax.axis_index("s")
    wid = sid * NC + cid
    t0 = wid * BPW

    iota = lax.iota(jnp.int32, 16)

    def ids_src(t):
        l = t // BT
        bt = t % BT
        return ids_h.at[l // 8, bt, l % 8]

    def fire_ids(t, k):
        pltpu.async_copy(ids_src(t), idsv[k], isem[k])
        pltpu.async_copy(pos_h.at[t // BT], posv[k], psem[k])

    def wait_ids(t, k):
        pltpu.make_async_copy(ids_src(t), idsv[k], isem[k]).wait()
        pltpu.make_async_copy(pos_h.at[t // BT], posv[k], psem[k]).wait()

    def fire_gather(k):
        pltpu.async_copy(tab_h.at[idsv[k]], rowsv[k], gsem[k])

    def wait_gather(k):
        pltpu.make_async_copy(tab_h.at[idsv[k]], rowsv[k], gsem[k]).wait()

    def out_dst(t):
        return out_h.at[t // BT, :, t % BT]

    def fire_out(t, k):
        pltpu.async_copy(obufv[k], out_dst(t), osem[k])

    def wait_out(t, k):
        pltpu.make_async_copy(obufv[k], out_dst(t), osem[k]).wait()

    rvecs = tuple(iota + (16 * j) for j in range(8))

    def compute(k, ko):
        def dbody(d, c):
            for u in range(2):
                du = d * 2 + u
                pv = posv[k][du, pl.ds(0, 16)]
                dvec = jnp.full((16,), 0, jnp.int32) + du
                for j in range(8):
                    val = plsc.load_gather(rowsv[k], [c[j], dvec]) + pv
                    obufv[ko][du // 8, du % 8, pl.ds(16 * j, 16)] = val
            return c

        lax.fori_loop(0, D // 2, dbody, rvecs)

    for k in range(NB - 1):
        fire_ids(t0 + k, k)
    for k in range(NB - 1):
        wait_ids(t0 + k, k)
        fire_gather(k)
    fire_ids(t0 + NB - 1, NB - 1)

    def stage(tt, kk, fire_next_gather, fire_next_ids, wait_prev_out):
        t = t0 + tt
        k = kk % NB
        ko = kk % 2
        kn = (kk + NB - 1) % NB
        wait_gather(k)
        if fire_next_gather:
            wait_ids(t + NB - 1, kn)
            fire_gather(kn)
        if wait_prev_out:
            wait_out(t - 2, ko)
        compute(k, ko)
        fire_out(t, ko)
        if fire_next_ids:
            fire_ids(t + NB, k)

    HEAD = 4
    TAIL = 4
    for tt in range(HEAD):
        stage(tt, tt, True, True, tt >= 2)

    def steady(i, c):
        tt = 4 * i + HEAD
        for u in range(4):
            stage(tt + u, u, True, True, True)
        return c

    lax.fori_loop(0, (BPW - HEAD - TAIL) // 4, steady, 0)

    for tt in range(BPW - TAIL, BPW):
        stage(tt, tt, tt + NB - 1 < BPW, tt + NB < BPW, True)
    wait_out(t0 + BPW - 2, 0)
    wait_out(t0 + BPW - 1, 1)


@jax.jit
def _run(ids4d, table, posb):
    mesh = plsc.VectorSubcoreMesh(
        core_axis_name="c", subcore_axis_name="s", num_cores=NC, num_subcores=NS
    )
    f = pl.kernel(
        _body,
        out_type=jax.ShapeDtypeStruct((L, 8, BT, 8, 128), jnp.float32),
        mesh=mesh,
        scratch_types=[
            [pltpu.VMEM((128,), jnp.int32) for _ in range(NB)],
            [pltpu.VMEM((128, D), jnp.float32) for _ in range(NB)],
            [pltpu.VMEM((8, 8, 128), jnp.float32) for _ in range(2)],
            [pltpu.VMEM((D, 16), jnp.float32) for _ in range(NB)],
            [pltpu.SemaphoreType.DMA for _ in range(NB)],
            [pltpu.SemaphoreType.DMA for _ in range(NB)],
            [pltpu.SemaphoreType.DMA for _ in range(NB)],
            [pltpu.SemaphoreType.DMA for _ in range(2)],
        ],
        compiler_params=pltpu.CompilerParams(use_tc_tiling_on_sc=False, needs_layout_passes=False),
    )
    return f(ids4d, table, posb)


def kernel(input_ids, token_table, pos_table):
    ids4d = (
        input_ids.astype(jnp.int32)
        .T.reshape(LT, 8, BT, 128)
        .transpose(0, 2, 1, 3)
    )
    posb = jnp.broadcast_to(
        pos_table[:L].T.reshape(1, D, L).transpose(2, 1, 0), (L, D, 16)
    )
    out5d = _run(ids4d, token_table, posb)
    return out5d.transpose(2, 4, 0, 1, 3).reshape(B, L, D)


# speedup vs baseline: 1.1397x; 1.1384x over previous
import jax
import jax.numpy as jnp
from jax import lax
from jax.experimental import pallas as pl
from jax.experimental.pallas import tpu as pltpu
from jax.experimental.pallas import tpu_sc as plsc

B = 4096
L = 200
D = 64
NC, NS = 2, 16
NW = NC * NS
LT = L // 8
BT = B // 128
BLOCKS = L * BT
BPW = BLOCKS // NW
NB = 4


def _body(ids_h, tab_h, pos_h, out_h, idsv, rowsv, obufv, posv, gsem, isem, psem, osem):
    cid = lax.axis_index("c")
    sid = lax.axis_index("s")
    wid = sid * NC + cid
    t0 = wid * BPW

    iota = lax.iota(jnp.int32, 16)

    def ids_src(t):
        l = t // BT
        bt = t % BT
        return ids_h.at[l // 8, bt, l % 8]

    def fire_ids(t, k):
        pltpu.async_copy(ids_src(t), idsv[k], isem[k])
        pltpu.async_copy(pos_h.at[t // BT], posv[k], psem[k])

    def wait_ids(t, k):
        pltpu.make_async_copy(ids_src(t), idsv[k], isem[k]).wait()
        pltpu.make_async_copy(pos_h.at[t // BT], posv[k], psem[k]).wait()

    def fire_gather(k):
        pltpu.async_copy(tab_h.at[idsv[k]], rowsv[k], gsem[k])

    def wait_gather(k):
        pltpu.make_async_copy(tab_h.at[idsv[k]], rowsv[k], gsem[k]).wait()

    def out_dst(t):
        return out_h.at[t // BT, :, t % BT]

    def fire_out(t, k):
        pltpu.async_copy(obufv[k], out_dst(t), osem[k])

    def wait_out(t, k):
        pltpu.make_async_copy(obufv[k], out_dst(t), osem[k]).wait()

    dt_vecs = tuple((iota + 16 * q) // 8 for q in range(4))
    d8_vecs = tuple((iota + 16 * q) % 8 for q in range(4))

    def compute(k, ko):
        pq = tuple(posv[k][pl.ds(16 * q, 16)] for q in range(4))

        def bbody(b, c):
            for u in range(2):
                bu = b * 2 + u
                bvec = jnp.full((16,), 0, jnp.int32) + bu
                for q in range(4):
                    val = rowsv[k][bu, pl.ds(16 * q, 16)] + pq[q]
                    plsc.store_scatter(
                        obufv[ko], [dt_vecs[q], d8_vecs[q], bvec], val
                    )
            return c

        lax.fori_loop(0, 64, bbody, 0)

    for k in range(NB - 1):
        fire_ids(t0 + k, k)
    for k in range(NB - 1):
        wait_ids(t0 + k, k)
        fire_gather(k)
    fire_ids(t0 + NB - 1, NB - 1)

    def stage(tt, kk, fire_next_gather, fire_next_ids, wait_prev_out):
        t = t0 + tt
        k = kk % NB
        ko = kk % 2
        kn = (kk + NB - 1) % NB
        wait_gather(k)
        if fire_next_gather:
            wait_ids(t + NB - 1, kn)
            fire_gather(kn)
        if wait_prev_out:
            wait_out(t - 2, ko)
        compute(k, ko)
        fire_out(t, ko)
        if fire_next_ids:
            fire_ids(t + NB, k)

    HEAD = 4
    TAIL = 4
    for tt in range(HEAD):
        stage(tt, tt, True, True, tt >= 2)

    def steady(i, c):
        tt = 4 * i + HEAD
        for u in range(4):
            stage(tt + u, u, True, True, True)
        return c

    lax.fori_loop(0, (BPW - HEAD - TAIL) // 4, steady, 0)

    for tt in range(BPW - TAIL, BPW):
        stage(tt, tt, tt + NB - 1 < BPW, tt + NB < BPW, True)
    wait_out(t0 + BPW - 2, 0)
    wait_out(t0 + BPW - 1, 1)


@jax.jit
def _run(ids4d, table, posb):
    mesh = plsc.VectorSubcoreMesh(
        core_axis_name="c", subcore_axis_name="s", num_cores=NC, num_subcores=NS
    )
    f = pl.kernel(
        _body,
        out_type=jax.ShapeDtypeStruct((L, 8, BT, 8, 128), jnp.float32),
        mesh=mesh,
        scratch_types=[
            [pltpu.VMEM((128,), jnp.int32) for _ in range(NB)],
            [pltpu.VMEM((128, D), jnp.float32) for _ in range(NB)],
            [pltpu.VMEM((8, 8, 128), jnp.float32) for _ in range(2)],
            [pltpu.VMEM((D,), jnp.float32) for _ in range(NB)],
            [pltpu.SemaphoreType.DMA for _ in range(NB)],
            [pltpu.SemaphoreType.DMA for _ in range(NB)],
            [pltpu.SemaphoreType.DMA for _ in range(NB)],
            [pltpu.SemaphoreType.DMA for _ in range(2)],
        ],
        compiler_params=pltpu.CompilerParams(use_tc_tiling_on_sc=False, needs_layout_passes=False),
    )
    return f(ids4d, table, posb)


def kernel(input_ids, token_table, pos_table):
    ids4d = (
        input_ids.astype(jnp.int32)
        .T.reshape(LT, 8, BT, 128)
        .transpose(0, 2, 1, 3)
    )
    posb = pos_table[:L]
    out5d = _run(ids4d, token_table, posb)
    return out5d.transpose(2, 4, 0, 1, 3).reshape(B, L, D)
